# Initial kernel scaffold; baseline (speedup 1.0000x reference)
import jax
import jax.numpy as jnp
from jax.experimental import pallas as pl


def kernel(samples, enc_weight, cent_weight):
    raise NotImplementedError("write your pallas kernel here")


# speedup vs baseline: 1.3645x; 1.3645x over previous
import jax
import jax.numpy as jnp
from jax.experimental import pallas as pl
from jax.experimental.pallas import tpu as pltpu

_BB = 1024
_DD = 2000


def _disthd_body(s_ref, e_ref, c_ref, o_ref, acc_ref, en_ref, wn_ref):
    j = pl.program_id(1)
    nd = pl.num_programs(1)

    enc = jax.lax.dot_general(
        s_ref[...], e_ref[...], (((1,), (1,)), ((), ())),
        preferred_element_type=jnp.float32)
    part = jax.lax.dot_general(
        enc, c_ref[...], (((1,), (0,)), ((), ())),
        preferred_element_type=jnp.float32)
    ensq = jnp.sum(enc * enc, axis=1, keepdims=True)
    csq = c_ref[...] * c_ref[...]
    ones = jnp.ones((1, csq.shape[0]), jnp.float32)
    wsq = jax.lax.dot_general(
        ones, csq, (((1,), (0,)), ((), ())),
        preferred_element_type=jnp.float32)

    @pl.when(j == 0)
    def _init():
        acc_ref[...] = part
        en_ref[...] = ensq
        wn_ref[...] = wsq

    @pl.when(j > 0)
    def _accum():
        acc_ref[...] += part
        en_ref[...] += ensq
        wn_ref[...] += wsq

    @pl.when(j == nd - 1)
    def _finish():
        en = jnp.maximum(jnp.sqrt(en_ref[...]), 1e-12)
        wn = jnp.maximum(jnp.sqrt(wn_ref[...]), 1e-12)
        o_ref[...] = acc_ref[...] / (en * wn)


def kernel(samples, enc_weight, cent_weight):
    B, F = samples.shape
    D = enc_weight.shape[0]
    C = cent_weight.shape[0]
    grid = (B // _BB, D // _DD)
    return pl.pallas_call(
        _disthd_body,
        grid=grid,
        in_specs=[
            pl.BlockSpec((_BB, F), lambda i, j: (i, 0)),
            pl.BlockSpec((_DD, F), lambda i, j: (j, 0)),
            pl.BlockSpec((_DD, C), lambda i, j: (j, 0)),
        ],
        out_specs=pl.BlockSpec((_BB, C), lambda i, j: (i, 0)),
        out_shape=jax.ShapeDtypeStruct((B, C), jnp.float32),
        scratch_shapes=[
            pltpu.VMEM((_BB, C), jnp.float32),
            pltpu.VMEM((_BB, 1), jnp.float32),
            pltpu.VMEM((1, C), jnp.float32),
        ],
        compiler_params=pltpu.CompilerParams(
            dimension_semantics=("parallel", "arbitrary")),
    )(samples, enc_weight, cent_weight.T)


# speedup vs baseline: 3.8031x; 2.7871x over previous
import jax
import jax.numpy as jnp
from jax.experimental import pallas as pl
from jax.experimental.pallas import tpu as pltpu

_DD = 2000
_BB = 2048


def _stage1_body(e_ref, c_ref, q_ref, p_ref, w_ref):
    j = pl.program_id(0)
    e = e_ref[...]
    c = c_ref[...]
    q = jax.lax.dot_general(e, e, (((0,), (0,)), ((), ())),
                            preferred_element_type=jnp.float32)
    p = jax.lax.dot_general(e, c, (((0,), (0,)), ((), ())),
                            preferred_element_type=jnp.float32)
    ones = jnp.ones((1, c.shape[0]), jnp.float32)
    w = jax.lax.dot_general(ones, c * c, (((1,), (0,)), ((), ())),
                            preferred_element_type=jnp.float32)

    @pl.when(j == 0)
    def _init():
        q_ref[...] = q
        p_ref[...] = p
        w_ref[...] = w

    @pl.when(j > 0)
    def _accum():
        q_ref[...] += q
        p_ref[...] += p
        w_ref[...] += w


def _stage2_body(s_ref, q_ref, p_ref, w_ref, o_ref):
    s = s_ref[...]
    sq = jax.lax.dot_general(s, q_ref[...], (((1,), (0,)), ((), ())),
                             preferred_element_type=jnp.float32)
    ensq = jnp.sum(sq * s, axis=1, keepdims=True)
    raw = jax.lax.dot_general(s, p_ref[...], (((1,), (0,)), ((), ())),
                              preferred_element_type=jnp.float32)
    en = jnp.maximum(jnp.sqrt(jnp.maximum(ensq, 0.0)), 1e-12)
    wn = jnp.maximum(jnp.sqrt(w_ref[...]), 1e-12)
    o_ref[...] = raw / (en * wn)


def kernel(samples, enc_weight, cent_weight):
    B, F = samples.shape
    D = enc_weight.shape[0]
    C = cent_weight.shape[0]

    q, p, wsq = pl.pallas_call(
        _stage1_body,
        grid=(D // _DD,),
        in_specs=[
            pl.BlockSpec((_DD, F), lambda j: (j, 0)),
            pl.BlockSpec((_DD, C), lambda j: (j, 0)),
        ],
        out_specs=[
            pl.BlockSpec((F, F), lambda j: (0, 0)),
            pl.BlockSpec((F, C), lambda j: (0, 0)),
            pl.BlockSpec((1, C), lambda j: (0, 0)),
        ],
        out_shape=[
            jax.ShapeDtypeStruct((F, F), jnp.float32),
            jax.ShapeDtypeStruct((F, C), jnp.float32),
            jax.ShapeDtypeStruct((1, C), jnp.float32),
        ],
        compiler_params=pltpu.CompilerParams(
            dimension_semantics=("arbitrary",)),
    )(enc_weight, cent_weight.T)

    return pl.pallas_call(
        _stage2_body,
        grid=(B // _BB,),
        in_specs=[
            pl.BlockSpec((_BB, F), lambda i: (i, 0)),
            pl.BlockSpec((F, F), lambda i: (0, 0)),
            pl.BlockSpec((F, C), lambda i: (0, 0)),
            pl.BlockSpec((1, C), lambda i: (0, 0)),
        ],
        out_specs=pl.BlockSpec((_BB, C), lambda i: (i, 0)),
        out_shape=jax.ShapeDtypeStruct((B, C), jnp.float32),
        compiler_params=pltpu.CompilerParams(
            dimension_semantics=("parallel",)),
    )(samples, q, p, wsq)


# speedup vs baseline: 4.8749x; 1.2818x over previous
import functools

import jax
import jax.numpy as jnp
from jax.experimental import pallas as pl
from jax.experimental.pallas import tpu as pltpu

_DD = 2048
_BB = 1024


def _body(nd, dd, d_total, s_ref, e_ref, c_ref, o_ref, q_ref, p_ref, w_ref):
    j = pl.program_id(0)

    @pl.when(j < nd)
    def _stage1():
        lim = d_total - j * dd
        e = e_ref[...]
        c = c_ref[...]
        e = jnp.where(jax.lax.broadcasted_iota(jnp.int32, e.shape, 0) < lim,
                      e, 0.0)
        c = jnp.where(jax.lax.broadcasted_iota(jnp.int32, c.shape, 1) < lim,
                      c, 0.0)
        q = jax.lax.dot_general(e, e, (((0,), (0,)), ((), ())),
                                preferred_element_type=jnp.float32)
        p = jax.lax.dot_general(e, c, (((0,), (1,)), ((), ())),
                                preferred_element_type=jnp.float32)
        ones = jnp.ones((1, dd), jnp.float32)
        w = jax.lax.dot_general(ones, c * c, (((1,), (1,)), ((), ())),
                                preferred_element_type=jnp.float32)

        @pl.when(j == 0)
        def _init():
            q_ref[...] = q
            p_ref[...] = p
            w_ref[...] = w

        @pl.when(j > 0)
        def _accum():
            q_ref[...] += q
            p_ref[...] += p
            w_ref[...] += w

    @pl.when(j >= nd)
    def _stage2():
        s = s_ref[...]
        sq = jax.lax.dot_general(s, q_ref[...], (((1,), (0,)), ((), ())),
                                 preferred_element_type=jnp.float32)
        ensq = jnp.sum(sq * s, axis=1, keepdims=True)
        raw = jax.lax.dot_general(s, p_ref[...], (((1,), (0,)), ((), ())),
                                  preferred_element_type=jnp.float32)
        en = jnp.maximum(jnp.sqrt(jnp.maximum(ensq, 0.0)), 1e-12)
        wn = jnp.maximum(jnp.sqrt(w_ref[...]), 1e-12)
        o_ref[...] = raw / (en * wn)


def kernel(samples, enc_weight, cent_weight):
    B, F = samples.shape
    D = enc_weight.shape[0]
    C = cent_weight.shape[0]
    nd = -(-D // _DD)
    nb = B // _BB

    return pl.pallas_call(
        functools.partial(_body, nd, _DD, D),
        grid=(nd + nb,),
        in_specs=[
            pl.BlockSpec((_BB, F), lambda j: (jnp.maximum(j - nd, 0), 0)),
            pl.BlockSpec((_DD, F), lambda j: (jnp.minimum(j, nd - 1), 0)),
            pl.BlockSpec((C, _DD), lambda j: (0, jnp.minimum(j, nd - 1))),
        ],
        out_specs=pl.BlockSpec((_BB, C), lambda j: (jnp.maximum(j - nd, 0), 0)),
        out_shape=jax.ShapeDtypeStruct((B, C), jnp.float32),
        scratch_shapes=[
            pltpu.VMEM((F, F), jnp.float32),
            pltpu.VMEM((F, C), jnp.float32),
            pltpu.VMEM((1, C), jnp.float32),
        ],
        compiler_params=pltpu.CompilerParams(
            dimension_semantics=("arbitrary",)),
    )(samples, enc_weight, cent_weight)


# speedup vs baseline: 4.9395x; 1.0132x over previous
import functools

import jax
import jax.numpy as jnp
from jax.experimental import pallas as pl
from jax.experimental.pallas import tpu as pltpu

_DD = 2048
_BB = 1024


def _body(nd, dd, d_total, s_ref, e_ref, c_ref, o_ref, q_ref, p_ref, w_ref):
    j = pl.program_id(0)

    @pl.when(j < nd)
    def _stage1():
        lim = d_total - j * dd
        e = e_ref[...]
        c = c_ref[...]
        e = jnp.where(jax.lax.broadcasted_iota(jnp.int32, e.shape, 0) < lim,
                      e, 0.0)
        c = jnp.where(jax.lax.broadcasted_iota(jnp.int32, c.shape, 1) < lim,
                      c, 0.0)
        e16 = e.astype(jnp.bfloat16)
        c16 = c.astype(jnp.bfloat16)
        q = jax.lax.dot_general(e16, e16, (((0,), (0,)), ((), ())),
                                preferred_element_type=jnp.float32)
        p = jax.lax.dot_general(e16, c16, (((0,), (1,)), ((), ())),
                                preferred_element_type=jnp.float32)
        ones = jnp.ones((1, dd), jnp.float32)
        w = jax.lax.dot_general(ones, c * c, (((1,), (1,)), ((), ())),
                                preferred_element_type=jnp.float32)

        @pl.when(j == 0)
        def _init():
            q_ref[...] = q
            p_ref[...] = p
            w_ref[...] = w

        @pl.when(j > 0)
        def _accum():
            q_ref[...] += q
            p_ref[...] += p
            w_ref[...] += w

    @pl.when(j >= nd)
    def _stage2():
        s = s_ref[...]
        sq = jax.lax.dot_general(s, q_ref[...], (((1,), (0,)), ((), ())),
                                 preferred_element_type=jnp.float32)
        ensq = jnp.sum(sq * s, axis=1, keepdims=True)
        raw = jax.lax.dot_general(s, p_ref[...], (((1,), (0,)), ((), ())),
                                  preferred_element_type=jnp.float32)
        en = jnp.maximum(jnp.sqrt(jnp.maximum(ensq, 0.0)), 1e-12)
        wn = jnp.maximum(jnp.sqrt(w_ref[...]), 1e-12)
        o_ref[...] = raw / (en * wn)


def kernel(samples, enc_weight, cent_weight):
    B, F = samples.shape
    D = enc_weight.shape[0]
    C = cent_weight.shape[0]
    nd = -(-D // _DD)
    nb = B // _BB

    return pl.pallas_call(
        functools.partial(_body, nd, _DD, D),
        grid=(nd + nb,),
        in_specs=[
            pl.BlockSpec((_BB, F), lambda j: (jnp.maximum(j - nd, 0), 0)),
            pl.BlockSpec((_DD, F), lambda j: (jnp.minimum(j, nd - 1), 0)),
            pl.BlockSpec((C, _DD), lambda j: (0, jnp.minimum(j, nd - 1))),
        ],
        out_specs=pl.BlockSpec((_BB, C), lambda j: (jnp.maximum(j - nd, 0), 0)),
        out_shape=jax.ShapeDtypeStruct((B, C), jnp.float32),
        scratch_shapes=[
            pltpu.VMEM((F, F), jnp.float32),
            pltpu.VMEM((F, C), jnp.float32),
            pltpu.VMEM((1, C), jnp.float32),
        ],
        compiler_params=pltpu.CompilerParams(
            dimension_semantics=("arbitrary",)),
    )(samples, enc_weight, cent_weight)
